# speedup vs baseline: 1.0707x; 1.0707x over previous
import functools

import jax
import jax.numpy as jnp
from jax.experimental import pallas as pl

N = 10000
HEADS = 5


def _mm_kernel(x_ref, w_ref, b_ref, o_ref):
    o_ref[...] = jnp.dot(x_ref[...], w_ref[...],
                         preferred_element_type=jnp.float32) + b_ref[...]


def _mm(x, w, b, bn=1000):
    n, k = x.shape
    m = w.shape[1]
    return pl.pallas_call(
        _mm_kernel,
        grid=(n // bn,),
        in_specs=[
            pl.BlockSpec((bn, k), lambda i: (i, 0)),
            pl.BlockSpec((k, m), lambda i: (0, 0)),
            pl.BlockSpec((m,), lambda i: (0,)),
        ],
        out_specs=pl.BlockSpec((bn, m), lambda i: (i, 0)),
        out_shape=jax.ShapeDtypeStruct((n, m), jnp.float32),
    )(x, w, b)


def _gat(x, edge_index, W, att_src, att_dst, bias, heads, out_dim, concat):
    num_nodes = x.shape[0]
    loop = jnp.arange(num_nodes, dtype=edge_index.dtype)
    src = jnp.concatenate([edge_index[0], loop])
    dst = jnp.concatenate([edge_index[1], loop])
    h = _mm(x, W, jnp.zeros((W.shape[1],), jnp.float32)).reshape(
        num_nodes, heads, out_dim)
    a_src = jnp.sum(h * att_src, axis=-1)
    a_dst = jnp.sum(h * att_dst, axis=-1)
    alpha = a_src[src] + a_dst[dst]
    alpha = jnp.where(alpha > 0, alpha, 0.2 * alpha)
    ex = jnp.exp(alpha)
    denom = jax.ops.segment_sum(ex, dst, num_segments=num_nodes)
    msg = h[src] * ex[:, :, None]
    out = jax.ops.segment_sum(msg, dst, num_segments=num_nodes)
    out = out / (denom[:, :, None] + 1e-16)
    if concat:
        out = out.reshape(num_nodes, heads * out_dim)
    else:
        out = jnp.mean(out, axis=1)
    return out + bias


def kernel(x, edge_index, emb_W, emb_b, W1, as1, ad1, b1, W2, as2, ad2, b2):
    h = _mm(x, emb_W, emb_b)
    h = _gat(h, edge_index, W1, as1, ad1, b1, HEADS, 64, True)
    h = jax.nn.elu(h)
    h = _gat(h, edge_index, W2, as2, ad2, b2, HEADS, 40, False)
    return jax.nn.log_softmax(h, axis=1)


# speedup vs baseline: 16.8068x; 15.6973x over previous
import functools

import jax
import jax.numpy as jnp
from jax import lax
from jax.experimental import pallas as pl
from jax.experimental.pallas import tpu as pltpu
from jax.experimental.pallas import tpu_sc as plsc

N = 10000
NPAD = 10240
IN = 128
HID = 64
HEADS = 5
C = 40

NC = 2
NS = 16
NWORK = NC * NS
CHUNK = 64
TW = 128

_f32 = jnp.float32
_i32 = jnp.int32

_L1_PASSES = (
    {"heads": (0, 1), "cmap": (0, 0, 0, 0, 1, 1, 1, 1)},
    {"heads": (2, 3), "cmap": (0, 0, 0, 0, 1, 1, 1, 1)},
    {"heads": (4, 0, 1, 2, 3),
     "cmap": (0, 0, 0, 0, 1, 2, 3, ("b", 4, 0, 8))},
)
_L2_PASSES = (
    {"heads": (0, 1), "cmap": (0, 0, ("b", 0, 1, 8), 1, 1, 0, 1, 1)},
    {"heads": (2, 3), "cmap": (0, 0, ("b", 0, 1, 8), 1, 1, 0, 1, 1)},
    {"heads": (4,), "cmap": (0, 0, 0, 0, 0, 0, 0, 0)},
)
_L1_DEN = (64, 80, 96, 112, 120)


def _t1_body(x_ref, ew_ref, eb_ref, w1_ref, ab1_ref, d2_ref,
             t0_ref, t1_ref, t2_ref, pq_ref):
    h0 = jnp.dot(x_ref[...], ew_ref[...], preferred_element_type=_f32)
    h0 = h0 + eb_ref[...]
    hf = jnp.dot(h0, w1_ref[...], preferred_element_type=_f32)
    bn = hf.shape[0]
    t0_ref[...] = hf[:, 0:128]
    t1_ref[...] = hf[:, 128:256]
    t2_ref[...] = jnp.concatenate(
        [hf[:, 256:320], jnp.zeros((bn, 64), _f32)], axis=1) + d2_ref[...]
    pq = jnp.dot(hf, ab1_ref[...], preferred_element_type=_f32)
    pq_ref[...] = jnp.concatenate([pq, jnp.zeros((bn, 112), _f32)], axis=1)


def _t1(xp, emb_W, emb_br, W1, AB1p, d2row):
    bn = 1024
    return pl.pallas_call(
        _t1_body,
        grid=(NPAD // bn,),
        in_specs=[
            pl.BlockSpec((bn, IN), lambda i: (i, 0)),
            pl.BlockSpec((IN, HID), lambda i: (0, 0)),
            pl.BlockSpec((1, HID), lambda i: (0, 0)),
            pl.BlockSpec((HID, HEADS * HID), lambda i: (0, 0)),
            pl.BlockSpec((HEADS * HID, 16), lambda i: (0, 0)),
            pl.BlockSpec((1, TW), lambda i: (0, 0)),
        ],
        out_specs=[
            pl.BlockSpec((bn, TW), lambda i: (i, 0)),
            pl.BlockSpec((bn, TW), lambda i: (i, 0)),
            pl.BlockSpec((bn, TW), lambda i: (i, 0)),
            pl.BlockSpec((bn, TW), lambda i: (i, 0)),
        ],
        out_shape=[
            jax.ShapeDtypeStruct((NPAD, TW), _f32),
            jax.ShapeDtypeStruct((NPAD, TW), _f32),
            jax.ShapeDtypeStruct((NPAD, TW), _f32),
            jax.ShapeDtypeStruct((NPAD, TW), _f32),
        ],
    )(xp, emb_W, emb_br, W1, AB1p, d2row)


def _t3_body(a0_ref, a1_ref, a2_ref, b1_ref, w2_ref, ab2_ref,
             dr_ref, t0_ref, t1_ref, t2_ref, pq_ref):
    na = a0_ref[0] + a0_ref[1]
    nb = a1_ref[0] + a1_ref[1]
    ncf = a2_ref[0] + a2_ref[1]
    num = jnp.concatenate([na, nb, ncf[:, 0:64]], axis=1)
    bn = num.shape[0]
    denb = jnp.concatenate(
        [jnp.broadcast_to(ncf[:, dc:dc + 1], (bn, HID)) for dc in _L1_DEN],
        axis=1)
    g = num / (denb + 1e-16) + b1_ref[...]
    g = jnp.where(g > 0, g, jnp.exp(jnp.minimum(g, 0.0)) - 1.0)
    hf = jnp.dot(g, w2_ref[...], preferred_element_type=_f32)
    hf = hf + dr_ref[...]
    t0_ref[...] = hf[:, 0:128]
    t1_ref[...] = hf[:, 128:256]
    t2_ref[...] = hf[:, 256:384]
    pq = jnp.dot(hf, ab2_ref[...], preferred_element_type=_f32)
    pq_ref[...] = jnp.concatenate([pq, jnp.zeros((bn, 112), _f32)], axis=1)


def _t3(acc10, acc11, acc12, b1r, W2p, AB2p, drow):
    bn = 1024
    return pl.pallas_call(
        _t3_body,
        grid=(NPAD // bn,),
        in_specs=[
            pl.BlockSpec((2, bn, TW), lambda i: (0, i, 0)),
            pl.BlockSpec((2, bn, TW), lambda i: (0, i, 0)),
            pl.BlockSpec((2, bn, TW), lambda i: (0, i, 0)),
            pl.BlockSpec((1, HEADS * HID), lambda i: (0, 0)),
            pl.BlockSpec((HEADS * HID, 384), lambda i: (0, 0)),
            pl.BlockSpec((384, 16), lambda i: (0, 0)),
            pl.BlockSpec((1, 384), lambda i: (0, 0)),
        ],
        out_specs=[
            pl.BlockSpec((bn, TW), lambda i: (i, 0)),
            pl.BlockSpec((bn, TW), lambda i: (i, 0)),
            pl.BlockSpec((bn, TW), lambda i: (i, 0)),
            pl.BlockSpec((bn, TW), lambda i: (i, 0)),
        ],
        out_shape=[
            jax.ShapeDtypeStruct((NPAD, TW), _f32),
            jax.ShapeDtypeStruct((NPAD, TW), _f32),
            jax.ShapeDtypeStruct((NPAD, TW), _f32),
            jax.ShapeDtypeStruct((NPAD, TW), _f32),
        ],
    )(acc10, acc11, acc12, b1r, W2p, AB2p, drow)


def _t5_body(a0_ref, a1_ref, a2_ref, b2_ref, o_ref):
    na = a0_ref[0] + a0_ref[1]
    nb = a1_ref[0] + a1_ref[1]
    nc = a2_ref[0] + a2_ref[1]
    bn = na.shape[0]
    z = jnp.zeros((bn, C), _f32)
    for h in range(HEADS):
        srcn = (na, na, nb, nb, nc)[h]
        off = (h % 2) * 40 if h < 4 else 0
        dc = 80 if h in (0, 2, 4) else 96
        dh = jnp.broadcast_to(srcn[:, dc:dc + 1], (bn, C))
        z = z + srcn[:, off:off + C] / (dh + 1e-16)
    z = z * (1.0 / HEADS) + b2_ref[...]
    m = jnp.max(z, axis=1, keepdims=True)
    e = jnp.exp(z - m)
    s = jnp.sum(e, axis=1, keepdims=True)
    o_ref[...] = z - m - jnp.log(s)


def _t5(acc20, acc21, acc22, b2r):
    bn = 1024
    return pl.pallas_call(
        _t5_body,
        grid=(NPAD // bn,),
        in_specs=[
            pl.BlockSpec((2, bn, TW), lambda i: (0, i, 0)),
            pl.BlockSpec((2, bn, TW), lambda i: (0, i, 0)),
            pl.BlockSpec((2, bn, TW), lambda i: (0, i, 0)),
            pl.BlockSpec((1, C), lambda i: (0, 0)),
        ],
        out_specs=pl.BlockSpec((bn, C), lambda i: (i, 0)),
        out_shape=jax.ShapeDtypeStruct((NPAD, C), _f32),
    )(acc20, acc21, acc22, b2r)


def _build_sc(passes, n_edges_tot):
    ntab = len(passes)
    ch_per_w = n_edges_tot // (NWORK * CHUNK)
    rows_per_tile = NPAD // NS
    mesh = plsc.VectorSubcoreMesh(core_axis_name="c", subcore_axis_name="s")

    out_type = ([jax.ShapeDtypeStruct((NC, NPAD, TW), _f32)
                 for _ in range(ntab)]
                + [jax.ShapeDtypeStruct((n_edges_tot,), _f32)
                   for _ in range(HEADS)])

    scratch = (
        [pltpu.VMEM_SHARED((NPAD, TW), _f32)]
        + [pltpu.VMEM((CHUNK,), _i32)] * 2
        + [pltpu.VMEM((CHUNK, TW), _f32)] * 2
        + [pltpu.VMEM((CHUNK,), _f32)] * HEADS
        + [pltpu.VMEM((CHUNK, TW), _f32)]
    )

    @functools.partial(
        pl.kernel, out_type=out_type, mesh=mesh, scratch_types=scratch,
        compiler_params=pltpu.CompilerParams(needs_layout_passes=False))
    def sck(*refs):
        src_r, dst_r = refs[0], refs[1]
        tabs = refs[2:2 + ntab]
        pq_r = refs[2 + ntab]
        accs = refs[3 + ntab:3 + 2 * ntab]
        exbs = refs[3 + 2 * ntab:3 + 2 * ntab + HEADS]
        sc = refs[3 + 2 * ntab + HEADS:]
        acc_s = sc[0]
        srcv, dstv = sc[1], sc[2]
        pr, qr = sc[3], sc[4]
        exw = sc[5:5 + HEADS]
        hr = sc[5 + HEADS]

        c = lax.axis_index("c")
        s = lax.axis_index("s")
        wid = s * NC + c
        tbase = s * rows_per_tile
        zv = jnp.zeros((16,), _f32)
        maskb = (lax.iota(_i32, 16) < 8).astype(_f32)

        def zhr(r, _):
            for j in range(TW // 16):
                hr[r, 16 * j:16 * (j + 1)] = zv
            return 0
        lax.fori_loop(0, CHUNK, zhr, 0)

        def zero_acc(k, _):
            pltpu.sync_copy(hr, acc_s.at[pl.ds(tbase + k * 64, 64)])
            return 0
        nzc = rows_per_tile // 64
        lax.fori_loop(0, nzc, zero_acc, 0)
        plsc.subcore_barrier()

        for p, cfg in enumerate(passes):
            heads = cfg["heads"]
            used = sorted(set(heads))

            def chunk_body(g, _, p=p, cfg=cfg, heads=heads, used=used):
                base = (wid * ch_per_w + g) * CHUNK
                pltpu.sync_copy(src_r.at[pl.ds(base, CHUNK)], srcv)
                pltpu.sync_copy(dst_r.at[pl.ds(base, CHUNK)], dstv)
                if p == 0:
                    pltpu.sync_copy(pq_r.at[srcv], pr)
                    pltpu.sync_copy(pq_r.at[dstv], qr)

                    def exgrp(i, _):
                        rows = lax.iota(_i32, 16) + 16 * i
                        for h in range(HEADS):
                            t = (plsc.load_gather(
                                     pr, [rows, jnp.full((16,), h, _i32)])
                                 + plsc.load_gather(
                                     qr,
                                     [rows, jnp.full((16,), 8 + h, _i32)]))
                            t = jnp.where(t > 0, t, 0.2 * t)
                            exw[h][pl.ds(16 * i, 16)] = jnp.exp(t)
                        return 0
                    lax.fori_loop(0, CHUNK // 16, exgrp, 0)
                    for h in range(HEADS):
                        pltpu.sync_copy(exw[h],
                                        exbs[h].at[pl.ds(base, CHUNK)])
                else:
                    for h in used:
                        pltpu.sync_copy(exbs[h].at[pl.ds(base, CHUNK)],
                                        exw[h])

                pltpu.sync_copy(tabs[p].at[srcv], hr)

                def srow(e, _, cfg=cfg, heads=heads, used=used):
                    ev = jnp.full((16,), e, _i32)
                    sp = {h: plsc.load_gather(exw[h], [ev]) for h in used}
                    for j, ent in enumerate(cfg["cmap"]):
                        sl = pl.ds(16 * j, 16)
                        if isinstance(ent, tuple):
                            _, k1, k2, m = ent
                            w = (sp[heads[k1]] * maskb
                                 + sp[heads[k2]] * (1.0 - maskb))
                        else:
                            w = sp[heads[ent]]
                        hr[e, sl] = hr[e, sl] * w
                    return 0
                lax.fori_loop(0, CHUNK, srow, 0)
                pltpu.sync_copy(hr, acc_s.at[dstv], add=True)
                return 0

            lax.fori_loop(0, ch_per_w, chunk_body, 0)
            plsc.subcore_barrier()

            def drain(r, _, p=p):
                o = tbase + r * 64
                pltpu.sync_copy(acc_s.at[pl.ds(o, 64)],
                                accs[p].at[c, pl.ds(o, 64)])
                return 0
            lax.fori_loop(0, nzc, drain, 0)
            if p + 1 < ntab:
                lax.fori_loop(0, CHUNK, zhr, 0)
                lax.fori_loop(0, nzc, zero_acc, 0)
                plsc.subcore_barrier()

    return sck


def kernel(x, edge_index, emb_W, emb_b, W1, as1, ad1, b1, W2, as2, ad2, b2):
    num_nodes = x.shape[0]
    e_raw = edge_index.shape[1]
    e_tot = e_raw + num_nodes
    et = ((e_tot + NWORK * CHUNK - 1) // (NWORK * CHUNK)) * (NWORK * CHUNK)

    loop = jnp.arange(num_nodes, dtype=_i32)
    padi = jnp.full((et - e_tot,), num_nodes, dtype=_i32)
    src = jnp.concatenate([edge_index[0].astype(_i32), loop, padi])
    dst = jnp.concatenate([edge_index[1].astype(_i32), loop, padi])

    xp = jnp.pad(x, ((0, NPAD - num_nodes), (0, 0)))
    emb_br = emb_b.reshape(1, HID)

    AB1p = jnp.zeros((HEADS * HID, 16), _f32)
    for h in range(HEADS):
        AB1p = AB1p.at[h * HID:(h + 1) * HID, h].set(as1[h])
        AB1p = AB1p.at[h * HID:(h + 1) * HID, 8 + h].set(ad1[h])
    d2row = jnp.zeros((1, TW), _f32)
    for dc in _L1_DEN:
        d2row = d2row.at[0, dc].set(1.0)

    col_off = (0, 40, 128, 168, 256)
    W2p = jnp.zeros((HEADS * HID, 3 * TW), _f32)
    AB2p = jnp.zeros((3 * TW, 16), _f32)
    for h in range(HEADS):
        o = col_off[h]
        W2p = W2p.at[:, o:o + C].set(W2[:, h * C:(h + 1) * C])
        AB2p = AB2p.at[o:o + C, h].set(as2[h])
        AB2p = AB2p.at[o:o + C, 8 + h].set(ad2[h])
    drow2 = jnp.zeros((1, 3 * TW), _f32)
    for dc in (80, 96, 128 + 80, 128 + 96, 256 + 80):
        drow2 = drow2.at[0, dc].set(1.0)

    b1r = b1.reshape(1, HEADS * HID)
    b2r = b2.reshape(1, C)

    t10, t11, t12, pq1 = _t1(xp, emb_W, emb_br, W1, AB1p, d2row)

    s1 = _build_sc(_L1_PASSES, et)
    acc10, acc11, acc12 = s1(src, dst, t10, t11, t12, pq1)[:3]

    t20, t21, t22, pq2 = _t3(acc10, acc11, acc12, b1r, W2p, AB2p, drow2)

    s2 = _build_sc(_L2_PASSES, et)
    acc20, acc21, acc22 = s2(src, dst, t20, t21, t22, pq2)[:3]

    return _t5(acc20, acc21, acc22, b2r)[:num_nodes]
